# baseline (speedup 1.0000x reference)
import jax
import jax.numpy as jnp
from jax.experimental import pallas as pl
from jax.experimental.pallas import tpu as pltpu

_B, _C, _H, _W = 4, 1280, 64, 64
_HW = _H * _W
_N, _RDIM = 16, 512
_CB = 128


def _body(rf_ref, m_ref, w_ref, b_ref, sp_ref, o_ref, acc_ref):
    ib = pl.program_id(1)

    @pl.when(ib == 0)
    def _compute_map():
        projT = jax.lax.dot_general(
            w_ref[...], rf_ref[...], (((1,), (1,)), ((), ())),
            preferred_element_type=jnp.float32)
        projT = projT + b_ref[...]
        mf = (m_ref[...] > 0.5).astype(jnp.float32)
        rows = []
        suffix = jnp.ones((1, _HW), jnp.float32)
        for i in reversed(range(_N)):
            mi = mf[i:i + 1, :]
            rows.append(mi * suffix)
            suffix = suffix * (1.0 - mi)
        a = jnp.concatenate(rows[::-1], axis=0)
        acc_ref[...] = jax.lax.dot_general(
            projT, a, (((1,), (0,)), ((), ())),
            preferred_element_type=jnp.float32)

    o_ref[...] = sp_ref[...] + acc_ref[...][None]


def kernel(spatial_features, region_features, region_masks, W_proj, b_proj):
    sp2 = spatial_features.reshape(_B, _C, _HW)
    m2 = region_masks.reshape(_N, _HW)
    b2 = b_proj.reshape(_C, 1)
    out = pl.pallas_call(
        _body,
        grid=(_C // _CB, _B),
        in_specs=[
            pl.BlockSpec((_N, _RDIM), lambda ic, ib: (0, 0)),
            pl.BlockSpec((_N, _HW), lambda ic, ib: (0, 0)),
            pl.BlockSpec((_CB, _RDIM), lambda ic, ib: (ic, 0)),
            pl.BlockSpec((_CB, 1), lambda ic, ib: (ic, 0)),
            pl.BlockSpec((1, _CB, _HW), lambda ic, ib: (ib, ic, 0)),
        ],
        out_specs=pl.BlockSpec((1, _CB, _HW), lambda ic, ib: (ib, ic, 0)),
        out_shape=jax.ShapeDtypeStruct((_B, _C, _HW), jnp.float32),
        scratch_shapes=[pltpu.VMEM((_CB, _HW), jnp.float32)],
    )(region_features, m2, W_proj, b2, sp2)
    return out.reshape(_B, _C, _H, _W)


# speedup vs baseline: 1.0553x; 1.0553x over previous
import jax
import jax.numpy as jnp
from jax.experimental import pallas as pl
from jax.experimental.pallas import tpu as pltpu

_B, _C, _H, _W = 4, 1280, 64, 64
_HW = _H * _W
_N, _RDIM = 16, 512
_CB = 640


def _body(rf_ref, m_ref, w_ref, b_ref, sp_ref, o_ref, acc_ref):
    ib = pl.program_id(1)

    @pl.when(ib == 0)
    def _compute_map():
        projT = jax.lax.dot_general(
            w_ref[...], rf_ref[...], (((1,), (1,)), ((), ())),
            preferred_element_type=jnp.float32)
        projT = projT + b_ref[...]
        mf = (m_ref[...] > 0.5).astype(jnp.float32)
        rows = []
        suffix = jnp.ones((1, _HW), jnp.float32)
        for i in reversed(range(_N)):
            mi = mf[i:i + 1, :]
            rows.append(mi * suffix)
            suffix = suffix * (1.0 - mi)
        a = jnp.concatenate(rows[::-1], axis=0)
        acc_ref[...] = jax.lax.dot_general(
            projT, a, (((1,), (0,)), ((), ())),
            preferred_element_type=jnp.float32)

    o_ref[...] = sp_ref[...] + acc_ref[...][None]


def kernel(spatial_features, region_features, region_masks, W_proj, b_proj):
    sp2 = spatial_features.reshape(_B, _C, _HW)
    m2 = region_masks.reshape(_N, _HW)
    b2 = b_proj.reshape(_C, 1)
    out = pl.pallas_call(
        _body,
        grid=(_C // _CB, _B),
        in_specs=[
            pl.BlockSpec((_N, _RDIM), lambda ic, ib: (0, 0)),
            pl.BlockSpec((_N, _HW), lambda ic, ib: (0, 0)),
            pl.BlockSpec((_CB, _RDIM), lambda ic, ib: (ic, 0)),
            pl.BlockSpec((_CB, 1), lambda ic, ib: (ic, 0)),
            pl.BlockSpec((1, _CB, _HW), lambda ic, ib: (ib, ic, 0)),
        ],
        out_specs=pl.BlockSpec((1, _CB, _HW), lambda ic, ib: (ib, ic, 0)),
        out_shape=jax.ShapeDtypeStruct((_B, _C, _HW), jnp.float32),
        scratch_shapes=[pltpu.VMEM((_CB, _HW), jnp.float32)],
    )(region_features, m2, W_proj, b2, sp2)
    return out.reshape(_B, _C, _H, _W)
